# baseline (speedup 1.0000x reference)
import dataclasses

import jax
import jax.numpy as jnp
from jax import lax
from jax.experimental import pallas as pl
from jax.experimental.pallas import tpu as pltpu
from jax.experimental.pallas import tpu_sc as plsc

N = 10000
NP = 10240
E = 320000
D = 128
NCORE = 2
NSUB = 16
CHUNK = 128
CPT = 80
HALF = CPT // 2
EPT = CPT * CHUNK
EPAD = NCORE * NSUB * EPT
NBUF = 2
DUMP = N + 8

_f32 = jnp.float32
_i32 = jnp.int32


def _mesh():
    return plsc.VectorSubcoreMesh(core_axis_name="c", subcore_axis_name="s")


def _sc_params():
    cp = pltpu.CompilerParams()
    if "needs_layout_passes" in pltpu.CompilerParams.__dataclass_fields__:
        cp = dataclasses.replace(cp, needs_layout_passes=False)
    return cp


def _deg_body(dst_hbm, id_hbm, z_hbm, out_hbm, didx, hist, idb, shist):
    c = lax.axis_index("c")
    s = lax.axis_index("s")
    pltpu.sync_copy(dst_hbm.at[s], didx)
    pltpu.sync_copy(id_hbm, idb)
    pltpu.sync_copy(z_hbm.at[pl.ds(0, 80)], hist)
    pltpu.sync_copy(z_hbm.at[pl.ds(0, 5)], shist.at[pl.ds(s * 5, 5)])
    plsc.subcore_barrier()
    ones = jnp.ones((16,), _f32)

    @pl.loop(0, 2 * CPT)
    def _(j):
        for k in range(CHUNK // 16):
            idxv = didx.at[j, pl.ds(k * 16, 16)][...]
            rowv = lax.shift_right_logical(idxv, 7)
            colv = lax.bitwise_and(idxv, 127)
            plsc.addupdate_scatter(hist, [rowv, colv], ones)

    pltpu.sync_copy(hist, shist.at[idb.at[0]], add=True)
    plsc.subcore_barrier()

    @pl.when(jnp.logical_and(c == 0, s == 0))
    def _():
        pltpu.sync_copy(shist, out_hbm)


def _deg_call(dst16, idrow, zeros128):
    f = pl.kernel(
        _deg_body,
        out_type=jax.ShapeDtypeStruct((80, 128), _f32),
        mesh=_mesh(),
        compiler_params=_sc_params(),
        scratch_types=[
            pltpu.VMEM((2 * CPT, CHUNK), _i32),
            pltpu.VMEM((80, 128), _f32),
            pltpu.VMEM((1, 80), _i32),
            pltpu.VMEM_SHARED((80, 128), _f32),
        ],
    )
    return f(dst16, idrow, zeros128)


def _agg_body(hs_hbm, src_hbm, dst_hbm, z_hbm, out_hbm, sidx, didx,
              gb0, acc, ssem):
    gbufs = [gb0]
    c = lax.axis_index("c")
    s = lax.axis_index("s")
    w = c * NSUB + s
    for k in range(5):
        pltpu.sync_copy(z_hbm, acc.at[pl.ds(s * 640 + k * 128, 128)])
    plsc.subcore_barrier()

    pltpu.sync_copy(src_hbm.at[w], sidx)
    pltpu.sync_copy(dst_hbm.at[w], didx)

    @pl.loop(0, CPT)
    def _(j):
        pltpu.sync_copy(hs_hbm.at[sidx.at[j]], gbufs[0])
        pltpu.sync_copy(gbufs[0], acc.at[didx.at[j]], add=True)

    plsc.subcore_barrier()

    @pl.when(c == 0)
    def _():
        pltpu.sync_copy(acc.at[pl.ds(s * 640, 640)], out_hbm.at[0].at[pl.ds(s * 640, 640)])

    @pl.when(c == 1)
    def _():
        pltpu.sync_copy(acc.at[pl.ds(s * 640, 640)], out_hbm.at[1].at[pl.ds(s * 640, 640)])


_AGG_KERNEL = None


def _agg_call(hs, srcp, dstp, zeros128):
    global _AGG_KERNEL
    if _AGG_KERNEL is None:
        _AGG_KERNEL = pl.kernel(
            _agg_body,
            out_type=jax.ShapeDtypeStruct((2, NP, D), _f32),
            mesh=_mesh(),
            scratch_types=[
                pltpu.VMEM((CPT, CHUNK), _i32),
                pltpu.VMEM((CPT, CHUNK), _i32),
                pltpu.VMEM((CHUNK, D), _f32),
                pltpu.VMEM_SHARED((NP, D), _f32),
                pltpu.SemaphoreType.DMA((NBUF,)),
            ],
        )
    return _AGG_KERNEL(hs, srcp, dstp, zeros128)


_BM = 1024
_GRID = NP // _BM
_DN = (((1,), (1,)), ((), ()))


def _tc1_body(x_ref, w1_ref, hist_ref, hs_ref, self_ref, dinv_ref):
    h1 = lax.dot_general(x_ref[...], w1_ref[...], _DN,
                         preferred_element_type=_f32)
    dinv = lax.rsqrt(hist_ref[...] + 1.0)
    dinv_ref[...] = dinv
    hs_ref[...] = dinv * h1
    self_ref[...] = (dinv * dinv) * h1


def _tc1(xp, W1, hist1):
    return pl.pallas_call(
        _tc1_body,
        grid=(_GRID,),
        in_specs=[
            pl.BlockSpec((_BM, D), lambda i: (i, 0)),
            pl.BlockSpec((D, D), lambda i: (0, 0)),
            pl.BlockSpec((_BM, 1), lambda i: (i, 0)),
        ],
        out_specs=[
            pl.BlockSpec((_BM, D), lambda i: (i, 0)),
            pl.BlockSpec((_BM, D), lambda i: (i, 0)),
            pl.BlockSpec((_BM, 1), lambda i: (i, 0)),
        ],
        out_shape=[
            jax.ShapeDtypeStruct((NP, D), _f32),
            jax.ShapeDtypeStruct((NP, D), _f32),
            jax.ShapeDtypeStruct((NP, 1), _f32),
        ],
    )(xp, W1, hist1)


def _tc2_body(agg_ref, self_ref, dinv_ref, b1_ref, w2_ref, hs2_ref, self2_ref):
    a = agg_ref[...]
    dinv = dinv_ref[...]
    x2 = jnp.maximum(dinv * (a[0] + a[1]) + self_ref[...] + b1_ref[...], 0.0)
    h2 = lax.dot_general(x2, w2_ref[...], _DN, preferred_element_type=_f32)
    hs2_ref[...] = dinv * h2
    self2_ref[...] = (dinv * dinv) * h2


def _tc2(agg1, self1, dinv, b1, W2):
    return pl.pallas_call(
        _tc2_body,
        grid=(_GRID,),
        in_specs=[
            pl.BlockSpec((2, _BM, D), lambda i: (0, i, 0)),
            pl.BlockSpec((_BM, D), lambda i: (i, 0)),
            pl.BlockSpec((_BM, 1), lambda i: (i, 0)),
            pl.BlockSpec((1, D), lambda i: (0, 0)),
            pl.BlockSpec((D, D), lambda i: (0, 0)),
        ],
        out_specs=[
            pl.BlockSpec((_BM, D), lambda i: (i, 0)),
            pl.BlockSpec((_BM, D), lambda i: (i, 0)),
        ],
        out_shape=[
            jax.ShapeDtypeStruct((NP, D), _f32),
            jax.ShapeDtypeStruct((NP, D), _f32),
        ],
    )(agg1, self1, dinv, b1, W2)


def _tc3_body(agg_ref, self2_ref, dinv_ref, b2_ref, wout_ref, bout_ref, out_ref):
    a = agg_ref[...]
    dinv = dinv_ref[...]
    x3 = jnp.maximum(dinv * (a[0] + a[1]) + self2_ref[...] + b2_ref[...], 0.0)
    out_ref[...] = lax.dot_general(x3, wout_ref[...], _DN,
                                   preferred_element_type=_f32) + bout_ref[...]


def _tc3(agg2, self2, dinv, b2, Wout, bout):
    nc = Wout.shape[0]
    return pl.pallas_call(
        _tc3_body,
        grid=(_GRID,),
        in_specs=[
            pl.BlockSpec((2, _BM, D), lambda i: (0, i, 0)),
            pl.BlockSpec((_BM, D), lambda i: (i, 0)),
            pl.BlockSpec((_BM, 1), lambda i: (i, 0)),
            pl.BlockSpec((1, D), lambda i: (0, 0)),
            pl.BlockSpec((nc, D), lambda i: (0, 0)),
            pl.BlockSpec((1, nc), lambda i: (0, 0)),
        ],
        out_specs=pl.BlockSpec((_BM, nc), lambda i: (i, 0)),
        out_shape=jax.ShapeDtypeStruct((NP, nc), _f32),
    )(agg2, self2, dinv, b2, Wout, bout)


@jax.jit
def kernel(x, edge_index, W1, b1, W2, b2, Wout, bout):
    src = edge_index[0].astype(_i32)
    dst = edge_index[1].astype(_i32)
    npad = EPAD - E
    srcp = jnp.concatenate([src, jnp.zeros((npad,), _i32)]).reshape(32, CPT, CHUNK)
    dstp = jnp.concatenate([dst, jnp.full((npad,), DUMP, _i32)]).reshape(32, CPT, CHUNK)
    dst16 = dstp.reshape(16, 2 * CPT, CHUNK)
    zeros128 = jnp.zeros((128, D), _f32)
    idrow = jnp.arange(80, dtype=_i32).reshape(1, 80)
    xp = jnp.pad(x, ((0, NP - N), (0, 0)))

    hist = _deg_call(dst16, idrow, zeros128)
    hist1 = hist.reshape(NP, 1)
    hs1, self1, dinv = _tc1(xp, W1, hist1)
    agg1 = _agg_call(hs1, srcp, dstp, zeros128)
    hs2, self2 = _tc2(agg1, self1, dinv, b1.reshape(1, D), W2)
    agg2 = _agg_call(hs2, srcp, dstp, zeros128)
    out = _tc3(agg2, self2, dinv, b2.reshape(1, D), Wout, bout.reshape(1, -1))
    return out[:N]


# speedup vs baseline: 1.5021x; 1.5021x over previous
import dataclasses

import jax
import jax.numpy as jnp
from jax import lax
from jax.experimental import pallas as pl
from jax.experimental.pallas import tpu as pltpu
from jax.experimental.pallas import tpu_sc as plsc

N = 10000
NP = 10240
E = 320000
D = 128
NCORE = 2
NSUB = 16
CHUNK = 128
CPT = 79
HALF = CPT // 2
EPT = CPT * CHUNK
EPAD = NCORE * NSUB * EPT
NBUF = 2
DUMP = N + 8

_f32 = jnp.float32
_i32 = jnp.int32


def _mesh():
    return plsc.VectorSubcoreMesh(core_axis_name="c", subcore_axis_name="s")


def _sc_params():
    cp = pltpu.CompilerParams()
    if "needs_layout_passes" in pltpu.CompilerParams.__dataclass_fields__:
        cp = dataclasses.replace(cp, needs_layout_passes=False)
    return cp


def _deg_body(dst_hbm, id_hbm, z_hbm, out_hbm, didx, hist, idb, shist):
    c = lax.axis_index("c")
    s = lax.axis_index("s")
    pltpu.sync_copy(dst_hbm.at[s], didx)
    pltpu.sync_copy(id_hbm, idb)
    pltpu.sync_copy(z_hbm.at[pl.ds(0, 80)], hist)
    pltpu.sync_copy(z_hbm.at[pl.ds(0, 5)], shist.at[pl.ds(s * 5, 5)])
    plsc.subcore_barrier()
    ones = jnp.ones((16,), _f32)

    @pl.loop(0, 2 * CPT)
    def _(j):
        for k in range(CHUNK // 16):
            idxv = didx.at[j, pl.ds(k * 16, 16)][...]
            rowv = lax.shift_right_logical(idxv, 7)
            colv = lax.bitwise_and(idxv, 127)
            plsc.addupdate_scatter(hist, [rowv, colv], ones)

    pltpu.sync_copy(hist, shist.at[idb.at[0]], add=True)
    plsc.subcore_barrier()

    @pl.when(jnp.logical_and(c == 0, s == 0))
    def _():
        pltpu.sync_copy(shist, out_hbm)


def _deg_call(dst16, idrow, zeros128):
    f = pl.kernel(
        _deg_body,
        out_type=jax.ShapeDtypeStruct((80, 128), _f32),
        mesh=_mesh(),
        compiler_params=_sc_params(),
        scratch_types=[
            pltpu.VMEM((2 * CPT, CHUNK), _i32),
            pltpu.VMEM((80, 128), _f32),
            pltpu.VMEM((1, 80), _i32),
            pltpu.VMEM_SHARED((80, 128), _f32),
        ],
    )
    return f(dst16, idrow, zeros128)


def _agg_body(hs_hbm, src_hbm, dst_hbm, z_hbm, out_hbm, sidx, didx,
              gb0, acc):
    gbufs = [gb0]
    c = lax.axis_index("c")
    s = lax.axis_index("s")
    w = c * NSUB + s
    for k in range(5):
        pltpu.sync_copy(z_hbm, acc.at[pl.ds(s * 640 + k * 128, 128)])
    plsc.subcore_barrier()

    pltpu.sync_copy(src_hbm.at[w], sidx)
    pltpu.sync_copy(dst_hbm.at[w], didx)

    @pl.loop(0, CPT)
    def _(j):
        pltpu.sync_copy(hs_hbm.at[sidx.at[j]], gbufs[0])
        pltpu.sync_copy(gbufs[0], acc.at[didx.at[j]], add=True)

    plsc.subcore_barrier()

    @pl.when(c == 0)
    def _():
        pltpu.sync_copy(acc.at[pl.ds(s * 640, 640)], out_hbm.at[0].at[pl.ds(s * 640, 640)])

    @pl.when(c == 1)
    def _():
        pltpu.sync_copy(acc.at[pl.ds(s * 640, 640)], out_hbm.at[1].at[pl.ds(s * 640, 640)])


_AGG_KERNEL = None


def _agg_call(hs, srcp, dstp, zeros128):
    global _AGG_KERNEL
    if _AGG_KERNEL is None:
        _AGG_KERNEL = pl.kernel(
            _agg_body,
            out_type=jax.ShapeDtypeStruct((2, NP, D), _f32),
            mesh=_mesh(),
            scratch_types=[
                pltpu.VMEM((CPT, CHUNK), _i32),
                pltpu.VMEM((CPT, CHUNK), _i32),
                pltpu.VMEM((CHUNK, D), _f32),
                pltpu.VMEM_SHARED((NP, D), _f32),
            ],
        )
    return _AGG_KERNEL(hs, srcp, dstp, zeros128)


_BM = 1024
_GRID = NP // _BM
_DN = (((1,), (1,)), ((), ()))


def _tc1_body(x_ref, w1_ref, hist_ref, hs_ref, self_ref, dinv_ref):
    h1 = lax.dot_general(x_ref[...], w1_ref[...], _DN,
                         preferred_element_type=_f32)
    dinv = lax.rsqrt(hist_ref[...] + 1.0)
    dinv_ref[...] = dinv
    hs_ref[...] = dinv * h1
    self_ref[...] = (dinv * dinv) * h1


def _tc1(xp, W1, hist1):
    return pl.pallas_call(
        _tc1_body,
        grid=(_GRID,),
        in_specs=[
            pl.BlockSpec((_BM, D), lambda i: (i, 0)),
            pl.BlockSpec((D, D), lambda i: (0, 0)),
            pl.BlockSpec((_BM, 1), lambda i: (i, 0)),
        ],
        out_specs=[
            pl.BlockSpec((_BM, D), lambda i: (i, 0)),
            pl.BlockSpec((_BM, D), lambda i: (i, 0)),
            pl.BlockSpec((_BM, 1), lambda i: (i, 0)),
        ],
        out_shape=[
            jax.ShapeDtypeStruct((NP, D), _f32),
            jax.ShapeDtypeStruct((NP, D), _f32),
            jax.ShapeDtypeStruct((NP, 1), _f32),
        ],
    )(xp, W1, hist1)


def _tc2_body(agg_ref, self_ref, dinv_ref, b1_ref, w2_ref, hs2_ref, self2_ref):
    a = agg_ref[...]
    dinv = dinv_ref[...]
    x2 = jnp.maximum(dinv * (a[0] + a[1]) + self_ref[...] + b1_ref[...], 0.0)
    h2 = lax.dot_general(x2, w2_ref[...], _DN, preferred_element_type=_f32)
    hs2_ref[...] = dinv * h2
    self2_ref[...] = (dinv * dinv) * h2


def _tc2(agg1, self1, dinv, b1, W2):
    return pl.pallas_call(
        _tc2_body,
        grid=(_GRID,),
        in_specs=[
            pl.BlockSpec((2, _BM, D), lambda i: (0, i, 0)),
            pl.BlockSpec((_BM, D), lambda i: (i, 0)),
            pl.BlockSpec((_BM, 1), lambda i: (i, 0)),
            pl.BlockSpec((1, D), lambda i: (0, 0)),
            pl.BlockSpec((D, D), lambda i: (0, 0)),
        ],
        out_specs=[
            pl.BlockSpec((_BM, D), lambda i: (i, 0)),
            pl.BlockSpec((_BM, D), lambda i: (i, 0)),
        ],
        out_shape=[
            jax.ShapeDtypeStruct((NP, D), _f32),
            jax.ShapeDtypeStruct((NP, D), _f32),
        ],
    )(agg1, self1, dinv, b1, W2)


def _tc3_body(agg_ref, self2_ref, dinv_ref, b2_ref, wout_ref, bout_ref, out_ref):
    a = agg_ref[...]
    dinv = dinv_ref[...]
    x3 = jnp.maximum(dinv * (a[0] + a[1]) + self2_ref[...] + b2_ref[...], 0.0)
    out_ref[...] = lax.dot_general(x3, wout_ref[...], _DN,
                                   preferred_element_type=_f32) + bout_ref[...]


def _tc3(agg2, self2, dinv, b2, Wout, bout):
    nc = Wout.shape[0]
    return pl.pallas_call(
        _tc3_body,
        grid=(_GRID,),
        in_specs=[
            pl.BlockSpec((2, _BM, D), lambda i: (0, i, 0)),
            pl.BlockSpec((_BM, D), lambda i: (i, 0)),
            pl.BlockSpec((_BM, 1), lambda i: (i, 0)),
            pl.BlockSpec((1, D), lambda i: (0, 0)),
            pl.BlockSpec((nc, D), lambda i: (0, 0)),
            pl.BlockSpec((1, nc), lambda i: (0, 0)),
        ],
        out_specs=pl.BlockSpec((_BM, nc), lambda i: (i, 0)),
        out_shape=jax.ShapeDtypeStruct((NP, nc), _f32),
    )(agg2, self2, dinv, b2, Wout, bout)


@jax.jit
def kernel(x, edge_index, W1, b1, W2, b2, Wout, bout):
    src = edge_index[0].astype(_i32)
    dst = edge_index[1].astype(_i32)
    npad = EPAD - E
    srcp = jnp.concatenate([src, jnp.zeros((npad,), _i32)]).reshape(32, CPT, CHUNK)
    dstp = jnp.concatenate([dst, jnp.full((npad,), DUMP, _i32)]).reshape(32, CPT, CHUNK)
    dst16 = dstp.reshape(16, 2 * CPT, CHUNK)
    zeros128 = jnp.zeros((128, D), _f32)
    idrow = jnp.arange(80, dtype=_i32).reshape(1, 80)
    xp = jnp.pad(x, ((0, NP - N), (0, 0)))

    hist = _deg_call(dst16, idrow, zeros128)
    hist1 = hist.reshape(NP, 1)
    hs1, self1, dinv = _tc1(xp, W1, hist1)
    agg1 = _agg_call(hs1, srcp, dstp, zeros128)
    hs2, self2 = _tc2(agg1, self1, dinv, b1.reshape(1, D), W2)
    agg2 = _agg_call(hs2, srcp, dstp, zeros128)
    out = _tc3(agg2, self2, dinv, b2.reshape(1, D), Wout, bout.reshape(1, -1))
    return out[:N]
